# Initial kernel scaffold; baseline (speedup 1.0000x reference)
import jax
import jax.numpy as jnp
from jax.experimental import pallas as pl


def kernel(x, edge_index, edge_attr, batch, Wq1, bq1, Wk1, bk1, Wv1, bv1, We1, Wskip1, bskip1, ln1_g, ln1_b, Wq2, bq2, Wk2, bk2, Wv2, bv2, We2, Wskip2, bskip2, ln2_g, ln2_b, W_mlp1, b_mlp1, W_mlp2, b_mlp2):
    raise NotImplementedError("write your pallas kernel here")


# speedup vs baseline: 6.2804x; 6.2804x over previous
import functools

import jax
import jax.numpy as jnp
import numpy as np
from jax import lax
from jax.experimental import pallas as pl
from jax.experimental.pallas import tpu as pltpu
from jax.experimental.pallas import tpu_sc as plsc

N = 10000
E = 320000
D_IN = 128
HID = 32
HEADS = 8
EDIM = 4
OUT_DIM = 10
G = 64
HO = HEADS * HID

NC = 2
NS = 16
L = 16

C1 = 32
C2 = 96
E_PAD = 331776
EPT1 = E_PAD // NS
EPT2 = E_PAD // (NC * NS)
NCH1 = EPT1 // C1
NCH2 = EPT2 // C2

STRIPE = 632
NROWS = NS * STRIPE
ROW1 = 128
ROW2 = 128
DSTRIPE = 80
DROWS = NS * DSTRIPE
INV_SQRT_OC = float(1.0 / np.sqrt(HID))

_sc_mesh = plsc.VectorSubcoreMesh(core_axis_name="c", subcore_axis_name="s")


def _ds16(r):
    return pl.ds(16 * r, 16)


def _hsum(x):
    li = lax.broadcasted_iota(jnp.int32, (L,), 0)
    for sh in (8, 4, 2, 1):
        x = x + x.at[li ^ sh].get(mode="promise_in_bounds")
    return x


def _edge1_body(qf, kf, vf, srcoff, dstoff, dsts, d8idx, loffb, eat, weh,
                out, out_den,
                gs_v, gd_v, sc_v, d8_v, lo_v, q_v, k_v, v_v, ea_v, we_v,
                w_v, d_v, acc_sh, den_sh, sem_q, sem_k, sem_v):
    c = lax.axis_index("c")
    s = lax.axis_index("s")

    pltpu.sync_copy(weh.at[c], we_v)

    def _zrow(i, carry):
        for j in range(ROW1 // L):
            w_v[i, _ds16(j)] = jnp.zeros((L,), jnp.float32)
            d_v[i, _ds16(j)] = jnp.zeros((L,), jnp.float32)
        return carry

    lax.fori_loop(0, C1, _zrow, 0)
    nfull = STRIPE // C1
    for t in range(nfull):
        pltpu.sync_copy(w_v, acc_sh.at[pl.ds(s * STRIPE + t * C1, C1)])
    rem = STRIPE - nfull * C1
    if rem:
        pltpu.sync_copy(w_v.at[pl.ds(0, rem)],
                        acc_sh.at[pl.ds(s * STRIPE + nfull * C1, rem)])
    pltpu.sync_copy(d_v, den_sh.at[pl.ds(s * DSTRIPE, C1)])
    pltpu.sync_copy(d_v.at[pl.ds(0, DSTRIPE - C1)],
                    den_sh.at[pl.ds(s * DSTRIPE + C1, DSTRIPE - C1)])
    plsc.subcore_barrier()

    li = lax.broadcasted_iota(jnp.int32, (L,), 0)

    def _chunk(ci, carry):
        base = s * EPT1 + ci * C1
        pltpu.sync_copy(srcoff.at[pl.ds(c * E_PAD + base, C1)], gs_v)
        pltpu.sync_copy(dstoff.at[pl.ds(c * E_PAD + base, C1)], gd_v)
        pltpu.sync_copy(dsts.at[pl.ds(base, C1)], sc_v)
        pltpu.sync_copy(d8idx.at[pl.ds(base, C1)], d8_v)
        pltpu.sync_copy(loffb.at[pl.ds(base, C1)], lo_v)
        pltpu.sync_copy(eat.at[pl.ds(base, C1)], ea_v)
        cq = pltpu.async_copy(qf.at[gd_v], q_v, sem_q)
        ck = pltpu.async_copy(kf.at[gs_v], k_v, sem_k)
        cv = pltpu.async_copy(vf.at[gs_v], v_v, sem_v)
        cq.wait()
        ck.wait()
        cv.wait()

        def _edge(e, ecarry):
            eav = ea_v[e, :]
            a0 = eav[0]
            a1 = eav[1]
            a2 = eav[2]
            a3 = eav[3]
            ps = []
            vs = []
            for r in range(8):
                er = (a0 * we_v[0, _ds16(r)] + a1 * we_v[1, _ds16(r)]
                      + a2 * we_v[2, _ds16(r)] + a3 * we_v[3, _ds16(r)])
                qv = q_v[e, _ds16(r)]
                kv = k_v[e, _ds16(r)] + er
                vv = v_v[e, _ds16(r)] + er
                ps.append(qv * kv)
                vs.append(vv)
            exl = jnp.zeros((L,), jnp.float32)
            for h in range(4):
                av = _hsum(ps[2 * h] + ps[2 * h + 1])
                exv = jnp.exp(av * INV_SQRT_OC)
                w_v[e, _ds16(2 * h)] = vs[2 * h] * exv
                w_v[e, _ds16(2 * h + 1)] = vs[2 * h + 1] * exv
                exl = jnp.where(li == h, exv, exl)
            for j in range(ROW1 // L):
                d_v[e, _ds16(j)] = jnp.zeros((L,), jnp.float32)
            lo = lo_v[e, :][0]
            d_v[e, pl.ds(lo, L)] = exl
            return ecarry

        lax.fori_loop(0, C1, _edge, 0)
        pltpu.sync_copy(w_v, acc_sh.at[sc_v], add=True)
        pltpu.sync_copy(d_v, den_sh.at[d8_v], add=True)
        return carry

    lax.fori_loop(0, NCH1, _chunk, 0)
    plsc.subcore_barrier()
    pltpu.sync_copy(acc_sh.at[pl.ds(s * STRIPE, STRIPE)],
                    out.at[c, pl.ds(s * STRIPE, STRIPE)])
    pltpu.sync_copy(den_sh.at[pl.ds(s * DSTRIPE, DSTRIPE)],
                    out_den.at[c, pl.ds(s * DSTRIPE, DSTRIPE)])


_edge1 = functools.partial(
    pl.kernel, _edge1_body, mesh=_sc_mesh,
    out_type=[
        jax.ShapeDtypeStruct((NC, NROWS, ROW1), jnp.float32),
        jax.ShapeDtypeStruct((NC, DROWS, 128), jnp.float32),
    ],
    scratch_types=[
        pltpu.VMEM((C1,), jnp.int32),
        pltpu.VMEM((C1,), jnp.int32),
        pltpu.VMEM((C1,), jnp.int32),
        pltpu.VMEM((C1,), jnp.int32),
        pltpu.VMEM((C1, L), jnp.int32),
        pltpu.VMEM((C1, 128), jnp.float32),
        pltpu.VMEM((C1, 128), jnp.float32),
        pltpu.VMEM((C1, 128), jnp.float32),
        pltpu.VMEM((C1, L), jnp.float32),
        pltpu.VMEM((EDIM, 128), jnp.float32),
        pltpu.VMEM((C1, ROW1), jnp.float32),
        pltpu.VMEM((C1, 128), jnp.float32),
        pltpu.VMEM_SHARED((NROWS, ROW1), jnp.float32),
        pltpu.VMEM_SHARED((DROWS, 128), jnp.float32),
        pltpu.SemaphoreType.DMA,
        pltpu.SemaphoreType.DMA,
        pltpu.SemaphoreType.DMA,
    ],
    compiler_params=pltpu.CompilerParams(needs_layout_passes=False),
)()


def _edge2_body(qkv, srcg, dstg, dsts, eat, we2, out,
                gs_v, gd_v, sc_v, dr_v, sr_v, ea_v, we_v, w_v, acc_sh,
                sem_d, sem_s):
    c = lax.axis_index("c")
    s = lax.axis_index("s")
    wid = s * NC + c

    pltpu.sync_copy(we2, we_v)

    def _zrow(i, carry):
        for j in range(ROW2 // L):
            w_v[i, _ds16(j)] = jnp.zeros((L,), jnp.float32)
        return carry

    lax.fori_loop(0, C2, _zrow, 0)
    nfull = STRIPE // C2
    for t in range(nfull):
        pltpu.sync_copy(w_v, acc_sh.at[pl.ds(s * STRIPE + t * C2, C2)])
    rem = STRIPE - nfull * C2
    if rem:
        pltpu.sync_copy(w_v.at[pl.ds(0, rem)],
                        acc_sh.at[pl.ds(s * STRIPE + nfull * C2, rem)])
    plsc.subcore_barrier()

    li = lax.broadcasted_iota(jnp.int32, (L,), 0)

    def _chunk(ci, carry):
        base = wid * EPT2 + ci * C2
        pltpu.sync_copy(srcg.at[pl.ds(base, C2)], gs_v)
        pltpu.sync_copy(dstg.at[pl.ds(base, C2)], gd_v)
        pltpu.sync_copy(dsts.at[pl.ds(base, C2)], sc_v)
        pltpu.sync_copy(eat.at[pl.ds(base, C2)], ea_v)
        cd = pltpu.async_copy(qkv.at[gd_v], dr_v, sem_d)
        cs = pltpu.async_copy(qkv.at[gs_v], sr_v, sem_s)
        cd.wait()
        cs.wait()

        def _edge(e, ecarry):
            eav = ea_v[e, :]
            a0 = eav[0]
            a1 = eav[1]
            a2 = eav[2]
            a3 = eav[3]
            ps = []
            vs = []
            for r in range(2):
                er = (a0 * we_v[0, _ds16(r)] + a1 * we_v[1, _ds16(r)]
                      + a2 * we_v[2, _ds16(r)] + a3 * we_v[3, _ds16(r)])
                qv = dr_v[e, _ds16(r)]
                kv = sr_v[e, _ds16(2 + r)] + er
                vv = sr_v[e, _ds16(4 + r)] + er
                ps.append(qv * kv)
                vs.append(vv)
            av = _hsum(ps[0] + ps[1])
            exv = jnp.exp(av * INV_SQRT_OC)
            w_v[e, _ds16(0)] = vs[0] * exv
            w_v[e, _ds16(1)] = vs[1] * exv
            w_v[e, _ds16(2)] = jnp.where(li == 0, exv, 0.0)
            return ecarry

        lax.fori_loop(0, C2, _edge, 0)
        pltpu.sync_copy(w_v, acc_sh.at[sc_v], add=True)
        return carry

    lax.fori_loop(0, NCH2, _chunk, 0)
    plsc.subcore_barrier()
    pltpu.sync_copy(acc_sh.at[pl.ds(s * STRIPE, STRIPE)],
                    out.at[c, pl.ds(s * STRIPE, STRIPE)])


_edge2 = functools.partial(
    pl.kernel, _edge2_body, mesh=_sc_mesh,
    out_type=jax.ShapeDtypeStruct((NC, NROWS, ROW2), jnp.float32),
    scratch_types=[
        pltpu.VMEM((C2,), jnp.int32),
        pltpu.VMEM((C2,), jnp.int32),
        pltpu.VMEM((C2,), jnp.int32),
        pltpu.VMEM((C2, 128), jnp.float32),
        pltpu.VMEM((C2, 128), jnp.float32),
        pltpu.VMEM((C2, L), jnp.float32),
        pltpu.VMEM((EDIM, HID), jnp.float32),
        pltpu.VMEM((C2, ROW2), jnp.float32),
        pltpu.VMEM_SHARED((NROWS, ROW2), jnp.float32),
        pltpu.SemaphoreType.DMA,
        pltpu.SemaphoreType.DMA,
    ],
    compiler_params=pltpu.CompilerParams(needs_layout_passes=False),
)()


_BLK = 1000


def _mm_body(x_ref, w_ref, b_ref, o_ref):
    o_ref[...] = jnp.dot(x_ref[...], w_ref[...],
                         preferred_element_type=jnp.float32) + b_ref[...]


def _mm(x, w, b):
    n, k = x.shape
    m = w.shape[1]
    return pl.pallas_call(
        _mm_body,
        grid=(n // _BLK,),
        in_specs=[
            pl.BlockSpec((_BLK, k), lambda i: (i, 0)),
            pl.BlockSpec((k, m), lambda i: (0, 0)),
            pl.BlockSpec((1, m), lambda i: (0, 0)),
        ],
        out_specs=pl.BlockSpec((_BLK, m), lambda i: (i, 0)),
        out_shape=jax.ShapeDtypeStruct((n, m), jnp.float32),
    )(x, w, b.reshape(1, m))


def _ln_relu(t, g, b):
    m = jnp.mean(t, axis=-1, keepdims=True)
    d = t - m
    v = jnp.mean(d * d, axis=-1, keepdims=True)
    return jnp.maximum(d * jax.lax.rsqrt(v + 1e-5) * g + b, 0.0)


def _fin1_body(a_ref, b_ref, da_ref, db_ref, skip_ref, rep_ref, g_ref,
               bb_ref, o_ref):
    num = jnp.concatenate([a_ref[...], b_ref[...]], axis=1)
    den8 = jnp.concatenate([da_ref[...], db_ref[...]], axis=1)
    den = jnp.dot(den8, rep_ref[...], preferred_element_type=jnp.float32)
    t = num / (den + 1e-16) + skip_ref[...]
    o_ref[...] = _ln_relu(t, g_ref[...], bb_ref[...])


def _fin1(agg_a, agg_b, den_a, den_b, skip, rep, g, b):
    return pl.pallas_call(
        _fin1_body,
        grid=(N // _BLK,),
        in_specs=[
            pl.BlockSpec((_BLK, ROW1), lambda i: (i, 0)),
            pl.BlockSpec((_BLK, ROW1), lambda i: (i, 0)),
            pl.BlockSpec((_BLK, 4), lambda i: (i, 0)),
            pl.BlockSpec((_BLK, 4), lambda i: (i, 0)),
            pl.BlockSpec((_BLK, HO), lambda i: (i, 0)),
            pl.BlockSpec((HEADS, HO), lambda i: (0, 0)),
            pl.BlockSpec((1, HO), lambda i: (0, 0)),
            pl.BlockSpec((1, HO), lambda i: (0, 0)),
        ],
        out_specs=pl.BlockSpec((_BLK, HO), lambda i: (i, 0)),
        out_shape=jax.ShapeDtypeStruct((N, HO), jnp.float32),
    )(agg_a, agg_b, den_a, den_b, skip, rep, g.reshape(1, HO),
      b.reshape(1, HO))


def _pool_body(a_ref, b_ref, skip_ref, g_ref, bb_ref, bat_ref,
               sums_ref, cnt_ref):
    i = pl.program_id(0)
    num = a_ref[:, :HID] + b_ref[:, :HID]
    den = a_ref[:, HID:HID + 1] + b_ref[:, HID:HID + 1]
    t = num / (den + 1e-16) + skip_ref[...]
    h2 = _ln_relu(t, g_ref[...], bb_ref[...])
    bat = bat_ref[0]
    gi = lax.broadcasted_iota(jnp.int32, (G, _BLK), 0).astype(jnp.float32)
    oh = (jnp.broadcast_to(bat, (G, _BLK)) == gi).astype(jnp.float32)

    @pl.when(i == 0)
    def _():
        sums_ref[...] = jnp.zeros_like(sums_ref)
        cnt_ref[...] = jnp.zeros_like(cnt_ref)

    sums_ref[...] += jnp.dot(oh, h2, preferred_element_type=jnp.float32)
    cnt_ref[...] += jnp.sum(oh, axis=1, keepdims=True)


def _pool(agg_a, agg_b, skip, g, b, bat):
    return pl.pallas_call(
        _pool_body,
        grid=(N // _BLK,),
        in_specs=[
            pl.BlockSpec((_BLK, ROW2), lambda i: (i, 0)),
            pl.BlockSpec((_BLK, ROW2), lambda i: (i, 0)),
            pl.BlockSpec((_BLK, HID), lambda i: (i, 0)),
            pl.BlockSpec((1, HID), lambda i: (0, 0)),
            pl.BlockSpec((1, HID), lambda i: (0, 0)),
            pl.BlockSpec((1, 1, _BLK), lambda i: (i, 0, 0)),
        ],
        out_specs=[
            pl.BlockSpec((G, HID), lambda i: (0, 0)),
            pl.BlockSpec((G, 1), lambda i: (0, 0)),
        ],
        out_shape=[
            jax.ShapeDtypeStruct((G, HID), jnp.float32),
            jax.ShapeDtypeStruct((G, 1), jnp.float32),
        ],
    )(agg_a, agg_b, skip, g.reshape(1, HID), b.reshape(1, HID), bat)


def _head_body(s_ref, c_ref, w1_ref, b1_ref, w2_ref, b2_ref, o_ref):
    pooled = s_ref[...] / jnp.maximum(c_ref[...], 1.0)
    z = jnp.maximum(
        jnp.dot(pooled, w1_ref[...], preferred_element_type=jnp.float32)
        + b1_ref[...], 0.0)
    o_ref[...] = jnp.dot(z, w2_ref[...],
                         preferred_element_type=jnp.float32) + b2_ref[...]


def _head(sums, cnt, w1, b1, w2, b2):
    h = HID // 2
    return pl.pallas_call(
        _head_body,
        out_shape=jax.ShapeDtypeStruct((G, OUT_DIM), jnp.float32),
    )(sums, cnt, w1, b1.reshape(1, h), w2, b2.reshape(1, OUT_DIM))


def kernel(x, edge_index, edge_attr, batch, Wq1, bq1, Wk1, bk1, Wv1, bv1,
           We1, Wskip1, bskip1, ln1_g, ln1_b, Wq2, bq2, Wk2, bk2, Wv2, bv2,
           We2, Wskip2, bskip2, ln2_g, ln2_b, W_mlp1, b_mlp1, W_mlp2, b_mlp2):
    src = edge_index[0]
    dst = edge_index[1]
    pad = E_PAD - E
    zpad = jnp.zeros((pad,), jnp.int32)
    src_p = jnp.concatenate([src, zpad])
    dst_g = jnp.concatenate([dst, zpad])
    dst_s = jnp.concatenate([dst, jnp.full((pad,), N, jnp.int32)])
    srcoff = jnp.concatenate([src_p, src_p + N])
    dstoff = jnp.concatenate([dst_g, dst_g + N])
    eat = jnp.concatenate([
        jnp.concatenate([edge_attr, jnp.zeros((pad, EDIM), jnp.float32)]),
        jnp.zeros((E_PAD, L - EDIM), jnp.float32)], axis=1)
    d8idx = dst_s // 8
    loffb = jnp.broadcast_to(((dst_s % 8) * L)[:, None],
                             (E_PAD, L))

    wall1 = jnp.concatenate([Wq1, Wk1, Wv1, Wskip1], axis=1)
    ball1 = jnp.concatenate([bq1, bk1, bv1, bskip1])
    proj1 = _mm(x, wall1, ball1)
    q1, k1, v1, skip1 = (proj1[:, :HO], proj1[:, HO:2 * HO],
                         proj1[:, 2 * HO:3 * HO], proj1[:, 3 * HO:])
    qf = jnp.concatenate([q1[:, :128], q1[:, 128:]], axis=0)
    kf = jnp.concatenate([k1[:, :128], k1[:, 128:]], axis=0)
    vf = jnp.concatenate([v1[:, :128], v1[:, 128:]], axis=0)
    weh = jnp.stack([We1[:, :128], We1[:, 128:]])

    agg1, den1 = _edge1(qf, kf, vf, srcoff, dstoff, dst_s, d8idx, loffb,
                        eat, weh)
    den1 = den1.reshape(NC, DROWS * 8, L)[:, :N, :4]
    rep = jnp.kron(jnp.eye(HEADS, dtype=jnp.float32),
                   jnp.ones((1, HID), jnp.float32))
    h1 = _fin1(agg1[0, :N], agg1[1, :N], den1[0], den1[1], skip1, rep,
               ln1_g, ln1_b)

    wall2 = jnp.concatenate([Wq2, Wk2, Wv2, Wskip2], axis=1)
    ball2 = jnp.concatenate([bq2, bk2, bv2, bskip2])
    proj2 = _mm(h1, wall2, ball2)
    q2, k2, v2, skip2 = (proj2[:, :HID], proj2[:, HID:2 * HID],
                         proj2[:, 2 * HID:3 * HID], proj2[:, 3 * HID:])

    agg2 = _edge2(proj2, src_p, dst_g, dst_s, eat, We2)

    bat = batch.astype(jnp.float32).reshape(N // _BLK, 1, _BLK)
    sums, cnt = _pool(agg2[0, :N], agg2[1, :N], skip2, ln2_g, ln2_b, bat)
    return _head(sums, cnt, W_mlp1, b_mlp1, W_mlp2, b_mlp2)


# speedup vs baseline: 8.8237x; 1.4050x over previous
import functools

import jax
import jax.numpy as jnp
import numpy as np
from jax import lax
from jax.experimental import pallas as pl
from jax.experimental.pallas import tpu as pltpu
from jax.experimental.pallas import tpu_sc as plsc

N = 10000
E = 320000
D_IN = 128
HID = 32
HEADS = 8
EDIM = 4
OUT_DIM = 10
G = 64
HO = HEADS * HID

NC = 2
NS = 16
L = 16

C1 = 16
C2 = 32
E_PAD = 331776
EPT1 = E_PAD // NS
EPT2 = E_PAD // (NC * NS)
NCH1 = EPT1 // C1
NCH2 = EPT2 // C2
NCHG1 = E_PAD // C1
NCHG2 = E_PAD // C2

STRIPE = 632
NROWS = NS * STRIPE
ROW1 = 128
ROW2 = 128
DSTRIPE = 80
DROWS = NS * DSTRIPE
INV_SQRT_OC = float(1.0 / np.sqrt(HID))

_sc_mesh = plsc.VectorSubcoreMesh(core_axis_name="c", subcore_axis_name="s")


def _ds16(r):
    return pl.ds(16 * r, 16)


def _hsum(x):
    li = lax.broadcasted_iota(jnp.int32, (L,), 0)
    for sh in (8, 4, 2, 1):
        x = x + x.at[li ^ sh].get(mode="promise_in_bounds")
    return x


def _edge1_body(qf, kvf, srcoff, dstoff, dsts, d8a, eat, weh, out, out_den,
                gs0, gs1, gs2, gs3, gd0, gd1, gd2, gd3,
                sc0, sc1, sc2, sc3, d80, d81, d82, d83,
                ea0, ea1, ea2, ea3,
                q0, q1, kv0, kv1, w_v, d_v, we_v, acc_sh, den_sh,
                si0, si1, si2, si3, sg0, sg1, ss):
    c = lax.axis_index("c")
    s = lax.axis_index("s")
    gsv = [gs0, gs1, gs2, gs3]
    gdv = [gd0, gd1, gd2, gd3]
    scv = [sc0, sc1, sc2, sc3]
    d8v = [d80, d81, d82, d83]
    ea = [ea0, ea1, ea2, ea3]
    qb = [q0, q1]
    kvb = [kv0, kv1]
    si = [si0, si1, si2, si3]
    sg = [sg0, sg1]

    pltpu.sync_copy(weh.at[c], we_v)

    def _zrow(i, carry):
        for j in range(ROW1 // L):
            w_v[i, _ds16(j)] = jnp.zeros((L,), jnp.float32)
            d_v[i, _ds16(j)] = jnp.zeros((L,), jnp.float32)
        return carry

    lax.fori_loop(0, C1, _zrow, 0)
    nfull = STRIPE // C1
    for t in range(nfull):
        pltpu.sync_copy(w_v, acc_sh.at[pl.ds(s * STRIPE + t * C1, C1)])
    rem = STRIPE - nfull * C1
    if rem:
        pltpu.sync_copy(w_v.at[pl.ds(0, rem)],
                        acc_sh.at[pl.ds(s * STRIPE + nfull * C1, rem)])
    dfull = DSTRIPE // C1
    for t in range(dfull):
        pltpu.sync_copy(d_v, den_sh.at[pl.ds(s * DSTRIPE + t * C1, C1)])
    drem = DSTRIPE - dfull * C1
    if drem:
        pltpu.sync_copy(d_v.at[pl.ds(0, drem)],
                        den_sh.at[pl.ds(s * DSTRIPE + dfull * C1, drem)])
    plsc.subcore_barrier()

    li = lax.broadcasted_iota(jnp.int32, (L,), 0)
    msk4 = li < 4

    def fire_idx(cid, slot):
        base = s * NCH1 * C1 + cid * C1
        pltpu.async_copy(srcoff.at[pl.ds(c * E_PAD + base, C1)],
                         gsv[slot], si[slot])
        pltpu.async_copy(dstoff.at[pl.ds(c * E_PAD + base, C1)],
                         gdv[slot], si[slot])
        pltpu.async_copy(dsts.at[pl.ds(base, C1)], scv[slot], si[slot])
        pltpu.async_copy(d8a.at[pl.ds(base, C1)], d8v[slot], si[slot])
        pltpu.async_copy(eat.at[pl.ds(base, C1)], ea[slot], si[slot])

    def wait_idx(slot):
        pltpu.make_async_copy(dsts.at[pl.ds(0, C1)], gsv[slot],
                              si[slot]).wait()
        pltpu.make_async_copy(dsts.at[pl.ds(0, C1)], gdv[slot],
                              si[slot]).wait()
        pltpu.make_async_copy(dsts.at[pl.ds(0, C1)], scv[slot],
                              si[slot]).wait()
        pltpu.make_async_copy(d8a.at[pl.ds(0, C1)], d8v[slot],
                              si[slot]).wait()
        pltpu.make_async_copy(eat.at[pl.ds(0, C1)], ea[slot],
                              si[slot]).wait()

    def fire_gathers(slot, p):
        pltpu.async_copy(qf.at[gdv[slot]], qb[p], sg[p])
        pltpu.async_copy(kvf.at[gsv[slot]], kvb[p], sg[p])

    def wait_gathers(p):
        pltpu.make_async_copy(qf.at[pl.ds(0, C1)], qb[p], sg[p]).wait()
        pltpu.make_async_copy(kvf.at[pl.ds(0, C1)], kvb[p], sg[p]).wait()

    def fire_scatters(slot):
        pltpu.async_copy(w_v, acc_sh.at[scv[slot]], ss, add=True)
        pltpu.async_copy(d_v, den_sh.at[d8v[slot]], ss, add=True)

    def drain_scatters():
        pltpu.make_async_copy(qf.at[pl.ds(0, C1)], w_v, ss).wait()
        pltpu.make_async_copy(qf.at[pl.ds(0, C1)], d_v, ss).wait()

    def compute(slot, p):
        q_v = qb[p]
        kv_v = kvb[p]
        ea_v = ea[slot]
        z = jnp.zeros((L,), jnp.float32)

        def _edge(e, exj):
            eav = ea_v[e, :]
            a0 = eav[0]
            a1 = eav[1]
            a2 = eav[2]
            a3 = eav[3]
            ps = []
            vs = []
            for r in range(8):
                er = (a0 * we_v[0, _ds16(r)] + a1 * we_v[1, _ds16(r)]
                      + a2 * we_v[2, _ds16(r)] + a3 * we_v[3, _ds16(r)])
                qv = q_v[e, _ds16(r)]
                kv = kv_v[e, _ds16(r)] + er
                vv = kv_v[e, _ds16(8 + r)] + er
                ps.append(qv * kv)
                vs.append(vv)
            exn = []
            for h in range(4):
                av = _hsum(ps[2 * h] + ps[2 * h + 1])
                exv = jnp.exp(av * INV_SQRT_OC)
                w_v[e, _ds16(2 * h)] = vs[2 * h] * exv
                w_v[e, _ds16(2 * h + 1)] = vs[2 * h + 1] * exv
                exn.append(jnp.where(li == e, exv, exj[h]))
            for j in range(ROW1 // L):
                d_v[e, _ds16(j)] = jnp.zeros((L,), jnp.float32)
            return tuple(exn)

        exj = lax.fori_loop(0, C1, _edge, (z, z, z, z))
        dstv = scv[slot][...]
        lov = (dstv % 8) * L
        for h in range(4):
            plsc.addupdate_scatter(d_v, [li, lov + h], exj[h])

    def _group(cid, carry):
        fire_idx(cid, 0)
        wait_idx(0)
        fire_gathers(0, 0)
        wait_gathers(0)
        compute(0, 0)
        fire_scatters(0)
        drain_scatters()
        return carry

    lax.fori_loop(0, NCH1, _group, 0)
    plsc.subcore_barrier()
    pltpu.sync_copy(acc_sh.at[pl.ds(s * STRIPE, STRIPE)],
                    out.at[c, pl.ds(s * STRIPE, STRIPE)])
    pltpu.sync_copy(den_sh.at[pl.ds(s * DSTRIPE, DSTRIPE)],
                    out_den.at[c, pl.ds(s * DSTRIPE, DSTRIPE)])


_edge1 = functools.partial(
    pl.kernel, _edge1_body, mesh=_sc_mesh,
    out_type=[
        jax.ShapeDtypeStruct((NC, NROWS, ROW1), jnp.float32),
        jax.ShapeDtypeStruct((NC, DROWS, 128), jnp.float32),
    ],
    scratch_types=[
        pltpu.VMEM((C1,), jnp.int32),
        pltpu.VMEM((C1,), jnp.int32),
        pltpu.VMEM((C1,), jnp.int32),
        pltpu.VMEM((C1,), jnp.int32),
        pltpu.VMEM((C1,), jnp.int32),
        pltpu.VMEM((C1,), jnp.int32),
        pltpu.VMEM((C1,), jnp.int32),
        pltpu.VMEM((C1,), jnp.int32),
        pltpu.VMEM((C1,), jnp.int32),
        pltpu.VMEM((C1,), jnp.int32),
        pltpu.VMEM((C1,), jnp.int32),
        pltpu.VMEM((C1,), jnp.int32),
        pltpu.VMEM((C1,), jnp.int32),
        pltpu.VMEM((C1,), jnp.int32),
        pltpu.VMEM((C1,), jnp.int32),
        pltpu.VMEM((C1,), jnp.int32),
        pltpu.VMEM((C1, L), jnp.float32),
        pltpu.VMEM((C1, L), jnp.float32),
        pltpu.VMEM((C1, L), jnp.float32),
        pltpu.VMEM((C1, L), jnp.float32),
        pltpu.VMEM((C1, 128), jnp.float32),
        pltpu.VMEM((C1, 128), jnp.float32),
        pltpu.VMEM((C1, 256), jnp.float32),
        pltpu.VMEM((C1, 256), jnp.float32),
        pltpu.VMEM((C1, ROW1), jnp.float32),
        pltpu.VMEM((C1, 128), jnp.float32),
        pltpu.VMEM((EDIM, 128), jnp.float32),
        pltpu.VMEM_SHARED((NROWS, ROW1), jnp.float32),
        pltpu.VMEM_SHARED((DROWS, 128), jnp.float32),
        pltpu.SemaphoreType.DMA,
        pltpu.SemaphoreType.DMA,
        pltpu.SemaphoreType.DMA,
        pltpu.SemaphoreType.DMA,
        pltpu.SemaphoreType.DMA,
        pltpu.SemaphoreType.DMA,
        pltpu.SemaphoreType.DMA,
    ],
    compiler_params=pltpu.CompilerParams(needs_layout_passes=False),
)()


def _edge2_body(qkv, srcg, dstg, dsts, eat, we2, out,
                gs0, gs1, gs2, gs3, gd0, gd1, gd2, gd3,
                sc0, sc1, sc2, sc3,
                ea0, ea1, ea2, ea3,
                d0, d1, s0, s1, w_v, we_v, acc_sh,
                si0, si1, si2, si3, sg0, sg1, ss):
    c = lax.axis_index("c")
    s = lax.axis_index("s")
    wid = s * NC + c
    gsv = [gs0, gs1, gs2, gs3]
    gdv = [gd0, gd1, gd2, gd3]
    scv = [sc0, sc1, sc2, sc3]
    ea = [ea0, ea1, ea2, ea3]
    db = [d0, d1]
    sb = [s0, s1]
    si = [si0, si1, si2, si3]
    sg = [sg0, sg1]

    pltpu.sync_copy(we2, we_v)

    def _zrow(i, carry):
        for j in range(ROW2 // L):
            w_v[i, _ds16(j)] = jnp.zeros((L,), jnp.float32)
        return carry

    lax.fori_loop(0, C2, _zrow, 0)
    nfull = STRIPE // C2
    for t in range(nfull):
        pltpu.sync_copy(w_v, acc_sh.at[pl.ds(s * STRIPE + t * C2, C2)])
    rem = STRIPE - nfull * C2
    if rem:
        pltpu.sync_copy(w_v.at[pl.ds(0, rem)],
                        acc_sh.at[pl.ds(s * STRIPE + nfull * C2, rem)])
    plsc.subcore_barrier()

    li = lax.broadcasted_iota(jnp.int32, (L,), 0)

    def fire_idx(cid, slot):
        base = wid * NCH2 * C2 + cid * C2
        pltpu.async_copy(srcg.at[pl.ds(base, C2)], gsv[slot], si[slot])
        pltpu.async_copy(dstg.at[pl.ds(base, C2)], gdv[slot], si[slot])
        pltpu.async_copy(dsts.at[pl.ds(base, C2)], scv[slot], si[slot])
        pltpu.async_copy(eat.at[pl.ds(base, C2)], ea[slot], si[slot])

    def wait_idx(slot):
        pltpu.make_async_copy(dsts.at[pl.ds(0, C2)], gsv[slot],
                              si[slot]).wait()
        pltpu.make_async_copy(dsts.at[pl.ds(0, C2)], gdv[slot],
                              si[slot]).wait()
        pltpu.make_async_copy(dsts.at[pl.ds(0, C2)], scv[slot],
                              si[slot]).wait()
        pltpu.make_async_copy(eat.at[pl.ds(0, C2)], ea[slot],
                              si[slot]).wait()

    def fire_gathers(slot, p):
        pltpu.async_copy(qkv.at[gdv[slot]], db[p], sg[p])
        pltpu.async_copy(qkv.at[gsv[slot]], sb[p], sg[p])

    def wait_gathers(p):
        pltpu.make_async_copy(qkv.at[pl.ds(0, C2)], db[p], sg[p]).wait()
        pltpu.make_async_copy(qkv.at[pl.ds(0, C2)], sb[p], sg[p]).wait()

    def fire_scatter(slot):
        pltpu.async_copy(w_v, acc_sh.at[scv[slot]], ss, add=True)

    def drain_scatter():
        pltpu.make_async_copy(qkv.at[pl.ds(0, C2)], w_v, ss).wait()

    def compute(slot, p):
        dr_v = db[p]
        sr_v = sb[p]
        ea_v = ea[slot]

        def _edge(e, ecarry):
            eav = ea_v[e, :]
            a0 = eav[0]
            a1 = eav[1]
            a2 = eav[2]
            a3 = eav[3]
            ps = []
            vs = []
            for r in range(2):
                er = (a0 * we_v[0, _ds16(r)] + a1 * we_v[1, _ds16(r)]
                      + a2 * we_v[2, _ds16(r)] + a3 * we_v[3, _ds16(r)])
                qv = dr_v[e, _ds16(r)]
                kv = sr_v[e, _ds16(2 + r)] + er
                vv = sr_v[e, _ds16(4 + r)] + er
                ps.append(qv * kv)
                vs.append(vv)
            av = _hsum(ps[0] + ps[1])
            exv = jnp.exp(av * INV_SQRT_OC)
            w_v[e, _ds16(0)] = vs[0] * exv
            w_v[e, _ds16(1)] = vs[1] * exv
            w_v[e, _ds16(2)] = jnp.where(li == 0, exv, 0.0)
            return ecarry

        lax.fori_loop(0, C2, _edge, 0)

    def _group(cid, carry):
        fire_idx(cid, 0)
        wait_idx(0)
        fire_gathers(0, 0)
        wait_gathers(0)
        compute(0, 0)
        fire_scatter(0)
        drain_scatter()
        return carry

    lax.fori_loop(0, NCH2, _group, 0)
    plsc.subcore_barrier()
    pltpu.sync_copy(acc_sh.at[pl.ds(s * STRIPE, STRIPE)],
                    out.at[c, pl.ds(s * STRIPE, STRIPE)])


_edge2 = functools.partial(
    pl.kernel, _edge2_body, mesh=_sc_mesh,
    out_type=jax.ShapeDtypeStruct((NC, NROWS, ROW2), jnp.float32),
    scratch_types=[
        pltpu.VMEM((C2,), jnp.int32),
        pltpu.VMEM((C2,), jnp.int32),
        pltpu.VMEM((C2,), jnp.int32),
        pltpu.VMEM((C2,), jnp.int32),
        pltpu.VMEM((C2,), jnp.int32),
        pltpu.VMEM((C2,), jnp.int32),
        pltpu.VMEM((C2,), jnp.int32),
        pltpu.VMEM((C2,), jnp.int32),
        pltpu.VMEM((C2,), jnp.int32),
        pltpu.VMEM((C2,), jnp.int32),
        pltpu.VMEM((C2,), jnp.int32),
        pltpu.VMEM((C2,), jnp.int32),
        pltpu.VMEM((C2, L), jnp.float32),
        pltpu.VMEM((C2, L), jnp.float32),
        pltpu.VMEM((C2, L), jnp.float32),
        pltpu.VMEM((C2, L), jnp.float32),
        pltpu.VMEM((C2, 128), jnp.float32),
        pltpu.VMEM((C2, 128), jnp.float32),
        pltpu.VMEM((C2, 128), jnp.float32),
        pltpu.VMEM((C2, 128), jnp.float32),
        pltpu.VMEM((C2, ROW2), jnp.float32),
        pltpu.VMEM((EDIM, HID), jnp.float32),
        pltpu.VMEM_SHARED((NROWS, ROW2), jnp.float32),
        pltpu.SemaphoreType.DMA,
        pltpu.SemaphoreType.DMA,
        pltpu.SemaphoreType.DMA,
        pltpu.SemaphoreType.DMA,
        pltpu.SemaphoreType.DMA,
        pltpu.SemaphoreType.DMA,
        pltpu.SemaphoreType.DMA,
    ],
    compiler_params=pltpu.CompilerParams(needs_layout_passes=False),
)()


_BLK = 1000


def _mm_body(x_ref, w_ref, b_ref, o_ref):
    o_ref[...] = jnp.dot(x_ref[...], w_ref[...],
                         preferred_element_type=jnp.float32) + b_ref[...]


def _mm(x, w, b):
    n, k = x.shape
    m = w.shape[1]
    return pl.pallas_call(
        _mm_body,
        grid=(n // _BLK,),
        in_specs=[
            pl.BlockSpec((_BLK, k), lambda i: (i, 0)),
            pl.BlockSpec((k, m), lambda i: (0, 0)),
            pl.BlockSpec((1, m), lambda i: (0, 0)),
        ],
        out_specs=pl.BlockSpec((_BLK, m), lambda i: (i, 0)),
        out_shape=jax.ShapeDtypeStruct((n, m), jnp.float32),
    )(x, w, b.reshape(1, m))


def _ln_relu(t, g, b):
    m = jnp.mean(t, axis=-1, keepdims=True)
    d = t - m
    v = jnp.mean(d * d, axis=-1, keepdims=True)
    return jnp.maximum(d * jax.lax.rsqrt(v + 1e-5) * g + b, 0.0)


def _fin1_body(a_ref, b_ref, da_ref, db_ref, skip_ref, rep_ref, g_ref,
               bb_ref, o_ref):
    num = jnp.concatenate([a_ref[...], b_ref[...]], axis=1)
    den8 = jnp.concatenate([da_ref[...], db_ref[...]], axis=1)
    den = jnp.dot(den8, rep_ref[...], preferred_element_type=jnp.float32)
    t = num / (den + 1e-16) + skip_ref[...]
    o_ref[...] = _ln_relu(t, g_ref[...], bb_ref[...])


def _fin1(agg_a, agg_b, den_a, den_b, skip, rep, g, b):
    return pl.pallas_call(
        _fin1_body,
        grid=(N // _BLK,),
        in_specs=[
            pl.BlockSpec((_BLK, ROW1), lambda i: (i, 0)),
            pl.BlockSpec((_BLK, ROW1), lambda i: (i, 0)),
            pl.BlockSpec((_BLK, 4), lambda i: (i, 0)),
            pl.BlockSpec((_BLK, 4), lambda i: (i, 0)),
            pl.BlockSpec((_BLK, HO), lambda i: (i, 0)),
            pl.BlockSpec((HEADS, HO), lambda i: (0, 0)),
            pl.BlockSpec((1, HO), lambda i: (0, 0)),
            pl.BlockSpec((1, HO), lambda i: (0, 0)),
        ],
        out_specs=pl.BlockSpec((_BLK, HO), lambda i: (i, 0)),
        out_shape=jax.ShapeDtypeStruct((N, HO), jnp.float32),
    )(agg_a, agg_b, den_a, den_b, skip, rep, g.reshape(1, HO),
      b.reshape(1, HO))


def _pool_body(a_ref, b_ref, skip_ref, g_ref, bb_ref, bat_ref,
               sums_ref, cnt_ref):
    i = pl.program_id(0)
    num = a_ref[:, :HID] + b_ref[:, :HID]
    den = a_ref[:, HID:HID + 1] + b_ref[:, HID:HID + 1]
    t = num / (den + 1e-16) + skip_ref[...]
    h2 = _ln_relu(t, g_ref[...], bb_ref[...])
    bat = bat_ref[0]
    gi = lax.broadcasted_iota(jnp.int32, (G, _BLK), 0).astype(jnp.float32)
    oh = (jnp.broadcast_to(bat, (G, _BLK)) == gi).astype(jnp.float32)

    @pl.when(i == 0)
    def _():
        sums_ref[...] = jnp.zeros_like(sums_ref)
        cnt_ref[...] = jnp.zeros_like(cnt_ref)

    sums_ref[...] += jnp.dot(oh, h2, preferred_element_type=jnp.float32)
    cnt_ref[...] += jnp.sum(oh, axis=1, keepdims=True)


def _pool(agg_a, agg_b, skip, g, b, bat):
    return pl.pallas_call(
        _pool_body,
        grid=(N // _BLK,),
        in_specs=[
            pl.BlockSpec((_BLK, ROW2), lambda i: (i, 0)),
            pl.BlockSpec((_BLK, ROW2), lambda i: (i, 0)),
            pl.BlockSpec((_BLK, HID), lambda i: (i, 0)),
            pl.BlockSpec((1, HID), lambda i: (0, 0)),
            pl.BlockSpec((1, HID), lambda i: (0, 0)),
            pl.BlockSpec((1, 1, _BLK), lambda i: (i, 0, 0)),
        ],
        out_specs=[
            pl.BlockSpec((G, HID), lambda i: (0, 0)),
            pl.BlockSpec((G, 1), lambda i: (0, 0)),
        ],
        out_shape=[
            jax.ShapeDtypeStruct((G, HID), jnp.float32),
            jax.ShapeDtypeStruct((G, 1), jnp.float32),
        ],
    )(agg_a, agg_b, skip, g.reshape(1, HID), b.reshape(1, HID), bat)


def _head_body(s_ref, c_ref, w1_ref, b1_ref, w2_ref, b2_ref, o_ref):
    pooled = s_ref[...] / jnp.maximum(c_ref[...], 1.0)
    z = jnp.maximum(
        jnp.dot(pooled, w1_ref[...], preferred_element_type=jnp.float32)
        + b1_ref[...], 0.0)
    o_ref[...] = jnp.dot(z, w2_ref[...],
                         preferred_element_type=jnp.float32) + b2_ref[...]


def _head(sums, cnt, w1, b1, w2, b2):
    h = HID // 2
    return pl.pallas_call(
        _head_body,
        out_shape=jax.ShapeDtypeStruct((G, OUT_DIM), jnp.float32),
    )(sums, cnt, w1, b1.reshape(1, h), w2, b2.reshape(1, OUT_DIM))


def kernel(x, edge_index, edge_attr, batch, Wq1, bq1, Wk1, bk1, Wv1, bv1,
           We1, Wskip1, bskip1, ln1_g, ln1_b, Wq2, bq2, Wk2, bk2, Wv2, bv2,
           We2, Wskip2, bskip2, ln2_g, ln2_b, W_mlp1, b_mlp1, W_mlp2, b_mlp2):
    src = edge_index[0]
    dst = edge_index[1]
    pad = E_PAD - E
    zpad = jnp.zeros((pad,), jnp.int32)
    src_p = jnp.concatenate([src, zpad])
    dst_g = jnp.concatenate([dst, zpad])
    dst_s = jnp.concatenate([dst, jnp.full((pad,), N, jnp.int32)])
    d8 = dst_s // 8
    loff_f = lax.bitcast_convert_type((dst_s % 8) * L, jnp.float32)
    eat = jnp.concatenate([
        jnp.concatenate([edge_attr, jnp.zeros((pad, EDIM), jnp.float32)]),
        loff_f[:, None],
        jnp.zeros((E_PAD, L - EDIM - 1), jnp.float32)], axis=1)

    srcoff = jnp.concatenate([src_p, src_p + N])
    dstoff = jnp.concatenate([dst_g, dst_g + N])

    wall1 = jnp.concatenate([Wq1, Wk1, Wv1, Wskip1], axis=1)
    ball1 = jnp.concatenate([bq1, bk1, bv1, bskip1])
    proj1 = _mm(x, wall1, ball1)
    q1, k1, v1, skip1 = (proj1[:, :HO], proj1[:, HO:2 * HO],
                         proj1[:, 2 * HO:3 * HO], proj1[:, 3 * HO:])
    qf = jnp.concatenate([q1[:, :128], q1[:, 128:]], axis=0)
    kvf = jnp.concatenate([
        jnp.concatenate([k1[:, :128], v1[:, :128]], axis=1),
        jnp.concatenate([k1[:, 128:], v1[:, 128:]], axis=1)], axis=0)
    weh = jnp.stack([We1[:, :128], We1[:, 128:]])

    agg1, den1 = _edge1(qf, kvf, srcoff, dstoff, dst_s, d8, eat, weh)
    den1 = den1.reshape(NC, DROWS * 8, L)[:, :N, :4]
    rep = jnp.kron(jnp.eye(HEADS, dtype=jnp.float32),
                   jnp.ones((1, HID), jnp.float32))
    h1 = _fin1(agg1[0, :N], agg1[1, :N], den1[0], den1[1], skip1, rep,
               ln1_g, ln1_b)

    wall2 = jnp.concatenate([Wq2, Wk2, Wv2, Wskip2], axis=1)
    ball2 = jnp.concatenate([bq2, bk2, bv2, bskip2])
    proj2 = _mm(h1, wall2, ball2)
    skip2 = proj2[:, 3 * HID:]

    agg2 = _edge2(proj2, src_p, dst_g, dst_s, eat, We2)

    bat = batch.astype(jnp.float32).reshape(N // _BLK, 1, _BLK)
    sums, cnt = _pool(agg2[0, :N], agg2[1, :N], skip2, ln2_g, ln2_b, bat)
    return _head(sums, cnt, W_mlp1, b_mlp1, W_mlp2, b_mlp2)


# speedup vs baseline: 16.9664x; 1.9228x over previous
import functools

import jax
import jax.numpy as jnp
import numpy as np
from jax import lax
from jax.experimental import pallas as pl
from jax.experimental.pallas import tpu as pltpu
from jax.experimental.pallas import tpu_sc as plsc

N = 10000
E = 320000
D_IN = 128
HID = 32
HEADS = 8
EDIM = 4
OUT_DIM = 10
G = 64
HO = HEADS * HID

NC = 2
NS = 16
L = 16

C1 = 16
C2 = 32
E_PAD = 331776
EPT1 = E_PAD // NS
EPT2 = E_PAD // (NC * NS)
NCH1 = EPT1 // C1
NCH2 = EPT2 // C2
NCHG1 = E_PAD // C1
NCHG2 = E_PAD // C2

STRIPE = 632
NROWS = NS * STRIPE
ROW1 = 128
ROW2 = 128
DSTRIPE = 80
DROWS = NS * DSTRIPE
INV_SQRT_OC = float(1.0 / np.sqrt(HID))

_sc_mesh = plsc.VectorSubcoreMesh(core_axis_name="c", subcore_axis_name="s")


def _ds16(r):
    return pl.ds(16 * r, 16)


def _hsum(x):
    li = lax.broadcasted_iota(jnp.int32, (L,), 0)
    for sh in (8, 4, 2, 1):
        x = x + x.at[li ^ sh].get(mode="promise_in_bounds")
    return x


def _edge1_body(qf, kvf, srcoff, dstoff, dsts, d8a, eat, weh, out, out_den,
                gs0, gs1, gs2, gs3, gd0, gd1, gd2, gd3,
                sc0, sc1, sc2, sc3, d80, d81, d82, d83,
                ea0, ea1, ea2, ea3,
                q0, q1, kv0, kv1, w_v, d_v, we_v, acc_sh, den_sh,
                si0, si1, si2, si3, sg0, sg1, ss):
    c = lax.axis_index("c")
    s = lax.axis_index("s")
    gsv = [gs0, gs1, gs2, gs3]
    gdv = [gd0, gd1, gd2, gd3]
    scv = [sc0, sc1, sc2, sc3]
    d8v = [d80, d81, d82, d83]
    ea = [ea0, ea1, ea2, ea3]
    qb = [q0, q1]
    kvb = [kv0, kv1]
    si = [si0, si1, si2, si3]
    sg = [sg0, sg1]

    pltpu.sync_copy(weh.at[c], we_v)

    def _zrow(i, carry):
        for j in range(ROW1 // L):
            w_v[i, _ds16(j)] = jnp.zeros((L,), jnp.float32)
            d_v[i, _ds16(j)] = jnp.zeros((L,), jnp.float32)
        return carry

    lax.fori_loop(0, C1, _zrow, 0)
    nfull = STRIPE // C1
    for t in range(nfull):
        pltpu.sync_copy(w_v, acc_sh.at[pl.ds(s * STRIPE + t * C1, C1)])
    rem = STRIPE - nfull * C1
    if rem:
        pltpu.sync_copy(w_v.at[pl.ds(0, rem)],
                        acc_sh.at[pl.ds(s * STRIPE + nfull * C1, rem)])
    dfull = DSTRIPE // C1
    for t in range(dfull):
        pltpu.sync_copy(d_v, den_sh.at[pl.ds(s * DSTRIPE + t * C1, C1)])
    drem = DSTRIPE - dfull * C1
    if drem:
        pltpu.sync_copy(d_v.at[pl.ds(0, drem)],
                        den_sh.at[pl.ds(s * DSTRIPE + dfull * C1, drem)])
    plsc.subcore_barrier()

    li = lax.broadcasted_iota(jnp.int32, (L,), 0)
    msk4 = li < 4

    def fire_idx(cid, slot):
        base = s * NCH1 * C1 + cid * C1
        pltpu.async_copy(srcoff.at[pl.ds(c * E_PAD + base, C1)],
                         gsv[slot], si[slot])
        pltpu.async_copy(dstoff.at[pl.ds(c * E_PAD + base, C1)],
                         gdv[slot], si[slot])
        pltpu.async_copy(dsts.at[pl.ds(base, C1)], scv[slot], si[slot])
        pltpu.async_copy(d8a.at[pl.ds(base, C1)], d8v[slot], si[slot])
        pltpu.async_copy(eat.at[pl.ds(base, C1)], ea[slot], si[slot])

    def wait_idx(slot):
        pltpu.make_async_copy(dsts.at[pl.ds(0, C1)], gsv[slot],
                              si[slot]).wait()
        pltpu.make_async_copy(dsts.at[pl.ds(0, C1)], gdv[slot],
                              si[slot]).wait()
        pltpu.make_async_copy(dsts.at[pl.ds(0, C1)], scv[slot],
                              si[slot]).wait()
        pltpu.make_async_copy(d8a.at[pl.ds(0, C1)], d8v[slot],
                              si[slot]).wait()
        pltpu.make_async_copy(eat.at[pl.ds(0, C1)], ea[slot],
                              si[slot]).wait()

    def fire_gathers(slot, p):
        pltpu.async_copy(qf.at[gdv[slot]], qb[p], sg[p])
        pltpu.async_copy(kvf.at[gsv[slot]], kvb[p], sg[p])

    def wait_gathers(p):
        pltpu.make_async_copy(qf.at[pl.ds(0, C1)], qb[p], sg[p]).wait()
        pltpu.make_async_copy(kvf.at[pl.ds(0, C1)], kvb[p], sg[p]).wait()

    def fire_scatters(slot):
        pltpu.async_copy(w_v, acc_sh.at[scv[slot]], ss, add=True)
        pltpu.async_copy(d_v, den_sh.at[d8v[slot]], ss, add=True)

    def drain_scatters():
        pltpu.make_async_copy(qf.at[pl.ds(0, C1)], w_v, ss).wait()
        pltpu.make_async_copy(qf.at[pl.ds(0, C1)], d_v, ss).wait()

    def compute(slot, p):
        q_v = qb[p]
        kv_v = kvb[p]
        ea_v = ea[slot]
        z = jnp.zeros((L,), jnp.float32)

        def _edge(e, exj):
            eav = ea_v[e, :]
            a0 = eav[0]
            a1 = eav[1]
            a2 = eav[2]
            a3 = eav[3]
            ps = []
            vs = []
            for r in range(8):
                er = (a0 * we_v[0, _ds16(r)] + a1 * we_v[1, _ds16(r)]
                      + a2 * we_v[2, _ds16(r)] + a3 * we_v[3, _ds16(r)])
                qv = q_v[e, _ds16(r)]
                kv = kv_v[e, _ds16(r)] + er
                vv = kv_v[e, _ds16(8 + r)] + er
                ps.append(qv * kv)
                vs.append(vv)
            exn = []
            for h in range(4):
                av = _hsum(ps[2 * h] + ps[2 * h + 1])
                exv = jnp.exp(av * INV_SQRT_OC)
                w_v[e, _ds16(2 * h)] = vs[2 * h] * exv
                w_v[e, _ds16(2 * h + 1)] = vs[2 * h + 1] * exv
                exn.append(jnp.where(li == e, exv, exj[h]))
            for j in range(ROW1 // L):
                d_v[e, _ds16(j)] = jnp.zeros((L,), jnp.float32)
            return tuple(exn)

        exj = lax.fori_loop(0, C1, _edge, (z, z, z, z))
        dstv = scv[slot][...]
        lov = (dstv % 8) * L
        for h in range(4):
            plsc.addupdate_scatter(d_v, [li, lov + h], exj[h])

    fire_idx(0, 0)
    fire_idx(1, 1)
    wait_idx(0)
    fire_gathers(0, 0)

    def _group(g, carry):
        for u in range(4):
            cid = 4 * g + u
            p = u & 1

            @pl.when(cid + 2 < NCH1)
            def _():
                fire_idx(cid + 2, (u + 2) & 3)

            @pl.when(cid + 1 < NCH1)
            def _():
                wait_idx((u + 1) & 3)
                fire_gathers((u + 1) & 3, p ^ 1)

            wait_gathers(p)
            if u == 0:
                @pl.when(g > 0)
                def _():
                    drain_scatters()
            else:
                drain_scatters()
            compute(u, p)
            fire_scatters(u)
        return carry

    lax.fori_loop(0, NCH1 // 4, _group, 0)
    drain_scatters()
    plsc.subcore_barrier()
    pltpu.sync_copy(acc_sh.at[pl.ds(s * STRIPE, STRIPE)],
                    out.at[c, pl.ds(s * STRIPE, STRIPE)])
    pltpu.sync_copy(den_sh.at[pl.ds(s * DSTRIPE, DSTRIPE)],
                    out_den.at[c, pl.ds(s * DSTRIPE, DSTRIPE)])


_edge1 = functools.partial(
    pl.kernel, _edge1_body, mesh=_sc_mesh,
    out_type=[
        jax.ShapeDtypeStruct((NC, NROWS, ROW1), jnp.float32),
        jax.ShapeDtypeStruct((NC, DROWS, 128), jnp.float32),
    ],
    scratch_types=[
        pltpu.VMEM((C1,), jnp.int32),
        pltpu.VMEM((C1,), jnp.int32),
        pltpu.VMEM((C1,), jnp.int32),
        pltpu.VMEM((C1,), jnp.int32),
        pltpu.VMEM((C1,), jnp.int32),
        pltpu.VMEM((C1,), jnp.int32),
        pltpu.VMEM((C1,), jnp.int32),
        pltpu.VMEM((C1,), jnp.int32),
        pltpu.VMEM((C1,), jnp.int32),
        pltpu.VMEM((C1,), jnp.int32),
        pltpu.VMEM((C1,), jnp.int32),
        pltpu.VMEM((C1,), jnp.int32),
        pltpu.VMEM((C1,), jnp.int32),
        pltpu.VMEM((C1,), jnp.int32),
        pltpu.VMEM((C1,), jnp.int32),
        pltpu.VMEM((C1,), jnp.int32),
        pltpu.VMEM((C1, L), jnp.float32),
        pltpu.VMEM((C1, L), jnp.float32),
        pltpu.VMEM((C1, L), jnp.float32),
        pltpu.VMEM((C1, L), jnp.float32),
        pltpu.VMEM((C1, 128), jnp.float32),
        pltpu.VMEM((C1, 128), jnp.float32),
        pltpu.VMEM((C1, 256), jnp.float32),
        pltpu.VMEM((C1, 256), jnp.float32),
        pltpu.VMEM((C1, ROW1), jnp.float32),
        pltpu.VMEM((C1, 128), jnp.float32),
        pltpu.VMEM((EDIM, 128), jnp.float32),
        pltpu.VMEM_SHARED((NROWS, ROW1), jnp.float32),
        pltpu.VMEM_SHARED((DROWS, 128), jnp.float32),
        pltpu.SemaphoreType.DMA,
        pltpu.SemaphoreType.DMA,
        pltpu.SemaphoreType.DMA,
        pltpu.SemaphoreType.DMA,
        pltpu.SemaphoreType.DMA,
        pltpu.SemaphoreType.DMA,
        pltpu.SemaphoreType.DMA,
    ],
    compiler_params=pltpu.CompilerParams(needs_layout_passes=False),
)()


def _edge2_body(qkv, srcg, dstg, dsts, eat, we2, out,
                gs0, gs1, gs2, gs3, gd0, gd1, gd2, gd3,
                sc0, sc1, sc2, sc3,
                ea0, ea1, ea2, ea3,
                d0, d1, s0, s1, w_v, we_v, acc_sh,
                si0, si1, si2, si3, sg0, sg1, ss):
    c = lax.axis_index("c")
    s = lax.axis_index("s")
    wid = s * NC + c
    gsv = [gs0, gs1, gs2, gs3]
    gdv = [gd0, gd1, gd2, gd3]
    scv = [sc0, sc1, sc2, sc3]
    ea = [ea0, ea1, ea2, ea3]
    db = [d0, d1]
    sb = [s0, s1]
    si = [si0, si1, si2, si3]
    sg = [sg0, sg1]

    pltpu.sync_copy(we2, we_v)

    def _zrow(i, carry):
        for j in range(ROW2 // L):
            w_v[i, _ds16(j)] = jnp.zeros((L,), jnp.float32)
        return carry

    lax.fori_loop(0, C2, _zrow, 0)
    nfull = STRIPE // C2
    for t in range(nfull):
        pltpu.sync_copy(w_v, acc_sh.at[pl.ds(s * STRIPE + t * C2, C2)])
    rem = STRIPE - nfull * C2
    if rem:
        pltpu.sync_copy(w_v.at[pl.ds(0, rem)],
                        acc_sh.at[pl.ds(s * STRIPE + nfull * C2, rem)])
    plsc.subcore_barrier()

    li = lax.broadcasted_iota(jnp.int32, (L,), 0)

    def fire_idx(cid, slot):
        base = wid * NCH2 * C2 + cid * C2
        pltpu.async_copy(srcg.at[pl.ds(base, C2)], gsv[slot], si[slot])
        pltpu.async_copy(dstg.at[pl.ds(base, C2)], gdv[slot], si[slot])
        pltpu.async_copy(dsts.at[pl.ds(base, C2)], scv[slot], si[slot])
        pltpu.async_copy(eat.at[pl.ds(base, C2)], ea[slot], si[slot])

    def wait_idx(slot):
        pltpu.make_async_copy(dsts.at[pl.ds(0, C2)], gsv[slot],
                              si[slot]).wait()
        pltpu.make_async_copy(dsts.at[pl.ds(0, C2)], gdv[slot],
                              si[slot]).wait()
        pltpu.make_async_copy(dsts.at[pl.ds(0, C2)], scv[slot],
                              si[slot]).wait()
        pltpu.make_async_copy(eat.at[pl.ds(0, C2)], ea[slot],
                              si[slot]).wait()

    def fire_gathers(slot, p):
        pltpu.async_copy(qkv.at[gdv[slot]], db[p], sg[p])
        pltpu.async_copy(qkv.at[gsv[slot]], sb[p], sg[p])

    def wait_gathers(p):
        pltpu.make_async_copy(qkv.at[pl.ds(0, C2)], db[p], sg[p]).wait()
        pltpu.make_async_copy(qkv.at[pl.ds(0, C2)], sb[p], sg[p]).wait()

    def fire_scatter(slot):
        pltpu.async_copy(w_v, acc_sh.at[scv[slot]], ss, add=True)

    def drain_scatter():
        pltpu.make_async_copy(qkv.at[pl.ds(0, C2)], w_v, ss).wait()

    def compute(slot, p):
        dr_v = db[p]
        sr_v = sb[p]
        ea_v = ea[slot]

        def _edge(e, ecarry):
            eav = ea_v[e, :]
            a0 = eav[0]
            a1 = eav[1]
            a2 = eav[2]
            a3 = eav[3]
            ps = []
            vs = []
            for r in range(2):
                er = (a0 * we_v[0, _ds16(r)] + a1 * we_v[1, _ds16(r)]
                      + a2 * we_v[2, _ds16(r)] + a3 * we_v[3, _ds16(r)])
                qv = dr_v[e, _ds16(r)]
                kv = sr_v[e, _ds16(2 + r)] + er
                vv = sr_v[e, _ds16(4 + r)] + er
                ps.append(qv * kv)
                vs.append(vv)
            av = _hsum(ps[0] + ps[1])
            exv = jnp.exp(av * INV_SQRT_OC)
            w_v[e, _ds16(0)] = vs[0] * exv
            w_v[e, _ds16(1)] = vs[1] * exv
            w_v[e, _ds16(2)] = jnp.where(li == 0, exv, 0.0)
            return ecarry

        lax.fori_loop(0, C2, _edge, 0)

    fire_idx(0, 0)
    fire_idx(1, 1)
    wait_idx(0)
    fire_gathers(0, 0)

    def _group(g, carry):
        for u in range(4):
            cid = 4 * g + u
            p = u & 1

            @pl.when(cid + 2 < NCH2)
            def _():
                fire_idx(cid + 2, (u + 2) & 3)

            @pl.when(cid + 1 < NCH2)
            def _():
                wait_idx((u + 1) & 3)
                fire_gathers((u + 1) & 3, p ^ 1)

            wait_gathers(p)
            if u == 0:
                @pl.when(g > 0)
                def _():
                    drain_scatter()
            else:
                drain_scatter()
            compute(u, p)
            fire_scatter(u)
        return carry

    lax.fori_loop(0, NCH2 // 4, _group, 0)
    drain_scatter()
    plsc.subcore_barrier()
    pltpu.sync_copy(acc_sh.at[pl.ds(s * STRIPE, STRIPE)],
                    out.at[c, pl.ds(s * STRIPE, STRIPE)])


_edge2 = functools.partial(
    pl.kernel, _edge2_body, mesh=_sc_mesh,
    out_type=jax.ShapeDtypeStruct((NC, NROWS, ROW2), jnp.float32),
    scratch_types=[
        pltpu.VMEM((C2,), jnp.int32),
        pltpu.VMEM((C2,), jnp.int32),
        pltpu.VMEM((C2,), jnp.int32),
        pltpu.VMEM((C2,), jnp.int32),
        pltpu.VMEM((C2,), jnp.int32),
        pltpu.VMEM((C2,), jnp.int32),
        pltpu.VMEM((C2,), jnp.int32),
        pltpu.VMEM((C2,), jnp.int32),
        pltpu.VMEM((C2,), jnp.int32),
        pltpu.VMEM((C2,), jnp.int32),
        pltpu.VMEM((C2,), jnp.int32),
        pltpu.VMEM((C2,), jnp.int32),
        pltpu.VMEM((C2, L), jnp.float32),
        pltpu.VMEM((C2, L), jnp.float32),
        pltpu.VMEM((C2, L), jnp.float32),
        pltpu.VMEM((C2, L), jnp.float32),
        pltpu.VMEM((C2, 128), jnp.float32),
        pltpu.VMEM((C2, 128), jnp.float32),
        pltpu.VMEM((C2, 128), jnp.float32),
        pltpu.VMEM((C2, 128), jnp.float32),
        pltpu.VMEM((C2, ROW2), jnp.float32),
        pltpu.VMEM((EDIM, HID), jnp.float32),
        pltpu.VMEM_SHARED((NROWS, ROW2), jnp.float32),
        pltpu.SemaphoreType.DMA,
        pltpu.SemaphoreType.DMA,
        pltpu.SemaphoreType.DMA,
        pltpu.SemaphoreType.DMA,
        pltpu.SemaphoreType.DMA,
        pltpu.SemaphoreType.DMA,
        pltpu.SemaphoreType.DMA,
    ],
    compiler_params=pltpu.CompilerParams(needs_layout_passes=False),
)()


_BLK = 1000


def _mm_body(x_ref, w_ref, b_ref, o_ref):
    o_ref[...] = jnp.dot(x_ref[...], w_ref[...],
                         preferred_element_type=jnp.float32) + b_ref[...]


def _mm(x, w, b):
    n, k = x.shape
    m = w.shape[1]
    return pl.pallas_call(
        _mm_body,
        grid=(n // _BLK,),
        in_specs=[
            pl.BlockSpec((_BLK, k), lambda i: (i, 0)),
            pl.BlockSpec((k, m), lambda i: (0, 0)),
            pl.BlockSpec((1, m), lambda i: (0, 0)),
        ],
        out_specs=pl.BlockSpec((_BLK, m), lambda i: (i, 0)),
        out_shape=jax.ShapeDtypeStruct((n, m), jnp.float32),
    )(x, w, b.reshape(1, m))


def _ln_relu(t, g, b):
    m = jnp.mean(t, axis=-1, keepdims=True)
    d = t - m
    v = jnp.mean(d * d, axis=-1, keepdims=True)
    return jnp.maximum(d * jax.lax.rsqrt(v + 1e-5) * g + b, 0.0)


def _fin1_body(a_ref, b_ref, da_ref, db_ref, skip_ref, rep_ref, g_ref,
               bb_ref, o_ref):
    num = jnp.concatenate([a_ref[...], b_ref[...]], axis=1)
    den8 = jnp.concatenate([da_ref[...], db_ref[...]], axis=1)
    den = jnp.dot(den8, rep_ref[...], preferred_element_type=jnp.float32)
    t = num / (den + 1e-16) + skip_ref[...]
    o_ref[...] = _ln_relu(t, g_ref[...], bb_ref[...])


def _fin1(agg_a, agg_b, den_a, den_b, skip, rep, g, b):
    return pl.pallas_call(
        _fin1_body,
        grid=(N // _BLK,),
        in_specs=[
            pl.BlockSpec((_BLK, ROW1), lambda i: (i, 0)),
            pl.BlockSpec((_BLK, ROW1), lambda i: (i, 0)),
            pl.BlockSpec((_BLK, 4), lambda i: (i, 0)),
            pl.BlockSpec((_BLK, 4), lambda i: (i, 0)),
            pl.BlockSpec((_BLK, HO), lambda i: (i, 0)),
            pl.BlockSpec((HEADS, HO), lambda i: (0, 0)),
            pl.BlockSpec((1, HO), lambda i: (0, 0)),
            pl.BlockSpec((1, HO), lambda i: (0, 0)),
        ],
        out_specs=pl.BlockSpec((_BLK, HO), lambda i: (i, 0)),
        out_shape=jax.ShapeDtypeStruct((N, HO), jnp.float32),
    )(agg_a, agg_b, den_a, den_b, skip, rep, g.reshape(1, HO),
      b.reshape(1, HO))


def _pool_body(a_ref, b_ref, skip_ref, g_ref, bb_ref, bat_ref,
               sums_ref, cnt_ref):
    i = pl.program_id(0)
    num = a_ref[:, :HID] + b_ref[:, :HID]
    den = a_ref[:, HID:HID + 1] + b_ref[:, HID:HID + 1]
    t = num / (den + 1e-16) + skip_ref[...]
    h2 = _ln_relu(t, g_ref[...], bb_ref[...])
    bat = bat_ref[0]
    gi = lax.broadcasted_iota(jnp.int32, (G, _BLK), 0).astype(jnp.float32)
    oh = (jnp.broadcast_to(bat, (G, _BLK)) == gi).astype(jnp.float32)

    @pl.when(i == 0)
    def _():
        sums_ref[...] = jnp.zeros_like(sums_ref)
        cnt_ref[...] = jnp.zeros_like(cnt_ref)

    sums_ref[...] += jnp.dot(oh, h2, preferred_element_type=jnp.float32)
    cnt_ref[...] += jnp.sum(oh, axis=1, keepdims=True)


def _pool(agg_a, agg_b, skip, g, b, bat):
    return pl.pallas_call(
        _pool_body,
        grid=(N // _BLK,),
        in_specs=[
            pl.BlockSpec((_BLK, ROW2), lambda i: (i, 0)),
            pl.BlockSpec((_BLK, ROW2), lambda i: (i, 0)),
            pl.BlockSpec((_BLK, HID), lambda i: (i, 0)),
            pl.BlockSpec((1, HID), lambda i: (0, 0)),
            pl.BlockSpec((1, HID), lambda i: (0, 0)),
            pl.BlockSpec((1, 1, _BLK), lambda i: (i, 0, 0)),
        ],
        out_specs=[
            pl.BlockSpec((G, HID), lambda i: (0, 0)),
            pl.BlockSpec((G, 1), lambda i: (0, 0)),
        ],
        out_shape=[
            jax.ShapeDtypeStruct((G, HID), jnp.float32),
            jax.ShapeDtypeStruct((G, 1), jnp.float32),
        ],
    )(agg_a, agg_b, skip, g.reshape(1, HID), b.reshape(1, HID), bat)


def _head_body(s_ref, c_ref, w1_ref, b1_ref, w2_ref, b2_ref, o_ref):
    pooled = s_ref[...] / jnp.maximum(c_ref[...], 1.0)
    z = jnp.maximum(
        jnp.dot(pooled, w1_ref[...], preferred_element_type=jnp.float32)
        + b1_ref[...], 0.0)
    o_ref[...] = jnp.dot(z, w2_ref[...],
                         preferred_element_type=jnp.float32) + b2_ref[...]


def _head(sums, cnt, w1, b1, w2, b2):
    h = HID // 2
    return pl.pallas_call(
        _head_body,
        out_shape=jax.ShapeDtypeStruct((G, OUT_DIM), jnp.float32),
    )(sums, cnt, w1, b1.reshape(1, h), w2, b2.reshape(1, OUT_DIM))


def kernel(x, edge_index, edge_attr, batch, Wq1, bq1, Wk1, bk1, Wv1, bv1,
           We1, Wskip1, bskip1, ln1_g, ln1_b, Wq2, bq2, Wk2, bk2, Wv2, bv2,
           We2, Wskip2, bskip2, ln2_g, ln2_b, W_mlp1, b_mlp1, W_mlp2, b_mlp2):
    src = edge_index[0]
    dst = edge_index[1]
    pad = E_PAD - E
    zpad = jnp.zeros((pad,), jnp.int32)
    src_p = jnp.concatenate([src, zpad])
    dst_g = jnp.concatenate([dst, zpad])
    dst_s = jnp.concatenate([dst, jnp.full((pad,), N, jnp.int32)])
    d8 = dst_s // 8
    loff_f = lax.bitcast_convert_type((dst_s % 8) * L, jnp.float32)
    eat = jnp.concatenate([
        jnp.concatenate([edge_attr, jnp.zeros((pad, EDIM), jnp.float32)]),
        loff_f[:, None],
        jnp.zeros((E_PAD, L - EDIM - 1), jnp.float32)], axis=1)

    srcoff = jnp.concatenate([src_p, src_p + N])
    dstoff = jnp.concatenate([dst_g, dst_g + N])

    wall1 = jnp.concatenate([Wq1, Wk1, Wv1, Wskip1], axis=1)
    ball1 = jnp.concatenate([bq1, bk1, bv1, bskip1])
    proj1 = _mm(x, wall1, ball1)
    q1, k1, v1, skip1 = (proj1[:, :HO], proj1[:, HO:2 * HO],
                         proj1[:, 2 * HO:3 * HO], proj1[:, 3 * HO:])
    qf = jnp.concatenate([q1[:, :128], q1[:, 128:]], axis=0)
    kvf = jnp.concatenate([
        jnp.concatenate([k1[:, :128], v1[:, :128]], axis=1),
        jnp.concatenate([k1[:, 128:], v1[:, 128:]], axis=1)], axis=0)
    weh = jnp.stack([We1[:, :128], We1[:, 128:]])

    agg1, den1 = _edge1(qf, kvf, srcoff, dstoff, dst_s, d8, eat, weh)
    den1 = den1.reshape(NC, DROWS * 8, L)[:, :N, :4]
    rep = jnp.kron(jnp.eye(HEADS, dtype=jnp.float32),
                   jnp.ones((1, HID), jnp.float32))
    h1 = _fin1(agg1[0, :N], agg1[1, :N], den1[0], den1[1], skip1, rep,
               ln1_g, ln1_b)

    wall2 = jnp.concatenate([Wq2, Wk2, Wv2, Wskip2], axis=1)
    ball2 = jnp.concatenate([bq2, bk2, bv2, bskip2])
    proj2 = _mm(h1, wall2, ball2)
    skip2 = proj2[:, 3 * HID:]

    agg2 = _edge2(proj2, src_p, dst_g, dst_s, eat, We2)

    bat = batch.astype(jnp.float32).reshape(N // _BLK, 1, _BLK)
    sums, cnt = _pool(agg2[0, :N], agg2[1, :N], skip2, ln2_g, ln2_b, bat)
    return _head(sums, cnt, W_mlp1, b_mlp1, W_mlp2, b_mlp2)
